# Initial kernel scaffold; baseline (speedup 1.0000x reference)
import jax
import jax.numpy as jnp
from jax.experimental import pallas as pl


def kernel(features, conv_w, conv_b, cls_w, cls_b, bbox_w, bbox_b, img_h, img_w):
    raise NotImplementedError("write your pallas kernel here")


# speedup vs baseline: 1.3561x; 1.3561x over previous
import jax
import jax.numpy as jnp
from jax import lax
from jax.experimental import pallas as pl
from jax.experimental.pallas import tpu as pltpu

H = 160
W = 160
A = 3
HW = H * W
N = HW * A
K = 1000
CH = 1024
NCH = N // CH
RT = 8
ROWS = H // RT
NMS_T = 0.7
MIN_SZ = 1e-3
CLAMP = 4.135166556742356
PAY = 16
BIG_NEG = -3.0e38
INT_MIN = -2147483648
MASK31 = 0x7FFFFFFF

_cparams = pltpu.CompilerParams(vmem_limit_bytes=120 * 1024 * 1024)


def _conv_head_kernel(x_ref, w9_ref, cb_ref, hw_ref, hb_ref, out_ref):
    cb = cb_ref[...]
    hw_m = hw_ref[...]
    hb = hb_ref[...]
    for rt in range(RT):
        acc = jnp.zeros((ROWS * W, 128), jnp.float32)
        for t in range(9):
            dh, dw = t // 3, t % 3
            sl = x_ref[rt * ROWS + dh: rt * ROWS + dh + ROWS, dw: dw + W, :]
            acc = acc + jnp.dot(sl.reshape(ROWS * W, 128), w9_ref[t],
                                preferred_element_type=jnp.float32)
        tact = jnp.maximum(acc + cb, 0.0)
        out_ref[rt * ROWS * W:(rt + 1) * ROWS * W, :] = (
            jnp.dot(tact, hw_m, preferred_element_type=jnp.float32) + hb)


def _f2key(x):
    b = lax.bitcast_convert_type(x, jnp.int32)
    return b ^ ((b >> 31) & MASK31)


def _dote(a, b):
    return jnp.dot(a, b, preferred_element_type=jnp.float32,
                   precision=lax.Precision.HIGHEST)


def _select_nms_kernel(sc_ref, pay_ref, bounds_ref, out_ref, buf_ref, s_ref):
    f32 = jnp.float32
    r_i = lax.broadcasted_iota(jnp.int32, (CH, CH), 0)
    c_i = lax.broadcasted_iota(jnp.int32, (CH, CH), 1)
    eye = (r_i == c_i).astype(f32)
    upper = (r_i < c_i).astype(f32)
    p_col = r_i.astype(f32)
    ones_row = jnp.full((1, CH), 1.0, f32)
    ones_col = jnp.full((CH, 1), 1.0, f32)

    def row_of(col):
        return _dote(ones_row, eye * col)

    def col_of(rowv):
        return _dote(eye * rowv, ones_col)

    keys = _f2key(sc_ref[...])
    candu = jnp.int32(0)
    for bit in range(31, -1, -1):
        m = INT_MIN if bit == 31 else jnp.int32(1 << bit)
        trial = candu | m
        scand = trial ^ INT_MIN
        cnt = jnp.sum((keys >= scand).astype(f32))
        candu = jnp.where(cnt >= K, trial, candu)
    t_key = candu ^ INT_MIN

    buf_ref[...] = jnp.zeros((CH, PAY), f32)
    kf = f32(K)

    def make_pass(strict):
        def body(c, countf):
            srow = sc_ref[pl.ds(c, 1), :]
            krow = _f2key(srow)
            if strict:
                mrowf = (krow > t_key).astype(f32)
            else:
                mrowf = (krow == t_key).astype(f32)
            prefix = _dote(mrowf, upper)
            if not strict:
                mrowf = mrowf * ((countf + prefix) < kf).astype(f32)
            pos = prefix + countf
            pt = (p_col == pos).astype(f32) * mrowf
            vals = pay_ref[c]
            buf_ref[...] += _dote(pt, vals)
            return countf + jnp.sum(mrowf)
        return body

    n1 = lax.fori_loop(0, NCH, make_pass(True), f32(0.0))
    lax.fori_loop(0, NCH, make_pass(False), n1)

    comp = buf_ref[...]
    slot = lax.broadcasted_iota(jnp.int32, (CH, 1), 0)
    valid = slot < K
    s_col = jnp.where(valid, comp[:, 0:1], BIG_NEG)
    i_col = jnp.where(valid, comp[:, 9:10], (N + slot).astype(f32))
    s_row = row_of(s_col)
    i_row = row_of(i_col)
    gt = (s_row > s_col) | ((s_row == s_col) & (i_row < i_col))
    rank = _dote(gt.astype(f32), ones_col)
    q = (row_of(rank) == p_col).astype(f32)
    srt = _dote(q, comp)

    aw = srt[:, 7:8]
    ah = srt[:, 8:9]
    dwc = jnp.minimum(srt[:, 3:4], CLAMP)
    dhc = jnp.minimum(srt[:, 4:5], CLAMP)
    px = srt[:, 1:2] * aw + srt[:, 5:6]
    py = srt[:, 2:3] * ah + srt[:, 6:7]
    pw = jnp.exp(dwc) * aw
    ph = jnp.exp(dhc) * ah
    bw = bounds_ref[0:1, 0:1]
    bh = bounds_ref[0:1, 1:2]
    x1 = jnp.minimum(jnp.maximum(px - 0.5 * pw, 0.0), bw)
    y1 = jnp.minimum(jnp.maximum(py - 0.5 * ph, 0.0), bh)
    x2 = jnp.minimum(jnp.maximum(px + 0.5 * pw, 0.0), bw)
    y2 = jnp.minimum(jnp.maximum(py + 0.5 * ph, 0.0), bh)
    keep0 = ((x2 - x1) >= MIN_SZ) & ((y2 - y1) >= MIN_SZ)

    x1r, y1r = row_of(x1), row_of(y1)
    x2r, y2r = row_of(x2), row_of(y2)
    area = (x2 - x1) * (y2 - y1)
    area_r = row_of(area)
    rb_n = 4
    rb_sz = CH // rb_n
    for rb in range(rb_n):
        s0 = rb * rb_sz
        xl = jnp.maximum(x1[s0:s0 + rb_sz], x1r)
        yt = jnp.maximum(y1[s0:s0 + rb_sz], y1r)
        xr = jnp.minimum(x2[s0:s0 + rb_sz], x2r)
        yb = jnp.minimum(y2[s0:s0 + rb_sz], y2r)
        inter = jnp.maximum(xr - xl, 0.0) * jnp.maximum(yb - yt, 0.0)
        union = area[s0:s0 + rb_sz] + area_r - inter
        iou = inter / (union + 1e-8)
        li = lax.broadcasted_iota(jnp.int32, (rb_sz, CH), 0) + s0
        lj = lax.broadcasted_iota(jnp.int32, (rb_sz, CH), 1)
        s_ref[s0:s0 + rb_sz, :] = ((iou > NMS_T) & (li < lj)).astype(f32)

    keep0f = keep0.astype(f32)

    def nms_cond(carry):
        return carry[1] > 0.5

    def nms_body(carry):
        kfv, _ = carry
        bad = _dote(row_of(kfv), s_ref[...])
        nk = jnp.where(col_of(bad) > 0.5, 0.0, keep0f)
        return nk, jnp.sum(jnp.abs(nk - kfv))

    keepf, _ = lax.while_loop(nms_cond, nms_body, (keep0f, f32(1.0)))

    f_col = jnp.where(keepf > 0.5, srt[:, 0:1], f32(-1e9))
    f_row = row_of(f_col)
    pos_c = slot.astype(f32)
    pos_r = row_of(pos_c)
    gt2 = (f_row > f_col) | ((f_row == f_col) & (pos_r < pos_c))
    rank2 = _dote(gt2.astype(f32), ones_col)
    q2 = (row_of(rank2) == p_col).astype(f32)
    m8 = jnp.concatenate(
        [x1, y1, x2, y2, f_col, jnp.zeros((CH, 3), f32)], axis=1)
    out_ref[...] = _dote(q2, m8)


def kernel(features, conv_w, conv_b, cls_w, cls_b, bbox_w, bbox_b,
           img_h, img_w):
    f32 = jnp.float32
    x = jnp.transpose(features[0], (1, 2, 0))
    x_pad = jnp.pad(x, ((1, 1), (1, 1), (0, 0)))
    w9 = jnp.transpose(conv_w, (2, 3, 1, 0)).reshape(9, 128, 128)
    head_w = jnp.concatenate(
        [cls_w[:, :, 0, 0].T, bbox_w[:, :, 0, 0].T,
         jnp.zeros((128, 1), f32)], axis=1)
    head_b = jnp.concatenate(
        [cls_b, bbox_b, jnp.zeros((1,), f32)]).reshape(1, 16)

    head = pl.pallas_call(
        _conv_head_kernel,
        out_shape=jax.ShapeDtypeStruct((HW, 16), f32),
        compiler_params=_cparams,
    )(x_pad, w9, conv_b.reshape(1, 128), head_w, head_b)

    scores = head[:, 0:3].reshape(N)
    deltas = head[:, 3:15].reshape(N, 4)

    scales = jnp.array([32.0], f32)
    ratios = jnp.array([0.5, 1.0, 2.0], f32)
    h_rat = jnp.sqrt(ratios)
    w_rat = 1.0 / h_rat
    ws = (w_rat[:, None] * scales[None, :]).reshape(-1)
    hs = (h_rat[:, None] * scales[None, :]).reshape(-1)
    cell = jnp.round(jnp.stack([-ws, -hs, ws, hs], axis=1) / 2.0)
    aw_t = cell[:, 2] - cell[:, 0]
    ah_t = cell[:, 3] - cell[:, 1]
    cx_t = cell[:, 0] + 0.5 * aw_t
    cy_t = cell[:, 1] + 0.5 * ah_t
    stride_h = jnp.asarray(img_h // H, f32)
    stride_w = jnp.asarray(img_w // W, f32)
    wg = jnp.broadcast_to(jnp.arange(W, dtype=f32)[None, :, None],
                          (H, W, A)).reshape(-1)
    hg = jnp.broadcast_to(jnp.arange(H, dtype=f32)[:, None, None],
                          (H, W, A)).reshape(-1)
    ag = jnp.broadcast_to(jnp.arange(A)[None, None, :], (H, W, A)).reshape(-1)
    cxg = wg * stride_w + cx_t[ag]
    cyg = hg * stride_h + cy_t[ag]
    awg = aw_t[ag]
    ahg = ah_t[ag]
    idxf = jnp.arange(N, dtype=f32)

    payload = jnp.concatenate(
        [scores[:, None], deltas, cxg[:, None], cyg[:, None],
         awg[:, None], ahg[:, None], idxf[:, None],
         jnp.zeros((N, 6), f32)], axis=1).reshape(NCH, CH, PAY)
    scores2d = scores.reshape(NCH, CH)
    bounds = jnp.zeros((1, 128), f32)
    bounds = bounds.at[0, 0].set(jnp.asarray(img_w, f32))
    bounds = bounds.at[0, 1].set(jnp.asarray(img_h, f32))

    final = pl.pallas_call(
        _select_nms_kernel,
        out_shape=jax.ShapeDtypeStruct((CH, 8), f32),
        scratch_shapes=[pltpu.VMEM((CH, PAY), f32),
                        pltpu.VMEM((CH, CH), f32)],
        compiler_params=_cparams,
    )(scores2d, payload, bounds)

    return final[:K, 0:4], final[:K, 4]


# speedup vs baseline: 2.0111x; 1.4830x over previous
import jax
import jax.numpy as jnp
from jax import lax
from jax.experimental import pallas as pl
from jax.experimental.pallas import tpu as pltpu

H = 160
W = 160
A = 3
HW = H * W
N = HW * A
K = 1000
CH = 1024
NCH = N // CH
RT = 8
ROWS = H // RT
NMS_T = 0.7
MIN_SZ = 1e-3
CLAMP = 4.135166556742356
PAY = 16
BIG_NEG = -3.0e38
INT_MIN = -2147483648
MASK31 = 0x7FFFFFFF

_cparams = pltpu.CompilerParams(vmem_limit_bytes=120 * 1024 * 1024)


def _conv_head_kernel(x_ref, w9_ref, cb_ref, hw_ref, hb_ref, out_ref):
    cb = cb_ref[...]
    hw_m = hw_ref[...]
    hb = hb_ref[...]
    for rt in range(RT):
        acc = jnp.zeros((ROWS * W, 128), jnp.float32)
        for t in range(9):
            dh, dw = t // 3, t % 3
            sl = x_ref[rt * ROWS + dh: rt * ROWS + dh + ROWS, dw: dw + W, :]
            acc = acc + jnp.dot(sl.reshape(ROWS * W, 128), w9_ref[t],
                                preferred_element_type=jnp.float32)
        tact = jnp.maximum(acc + cb, 0.0)
        out_ref[rt * ROWS * W:(rt + 1) * ROWS * W, :] = (
            jnp.dot(tact, hw_m, preferred_element_type=jnp.float32) + hb)


def _f2key(x):
    b = lax.bitcast_convert_type(x, jnp.int32)
    return b ^ ((b >> 31) & MASK31)


def _dote(a, b):
    return jnp.dot(a, b, preferred_element_type=jnp.float32,
                   precision=lax.Precision.HIGHEST)


def _select_nms_kernel(sc_ref, pay_ref, bounds_ref, out_ref, buf_ref, s_ref,
                       pos_ref):
    f32 = jnp.float32
    r_i = lax.broadcasted_iota(jnp.int32, (CH, CH), 0)
    c_i = lax.broadcasted_iota(jnp.int32, (CH, CH), 1)
    eye = (r_i == c_i).astype(f32)
    upper = (r_i < c_i).astype(f32)
    p_col = r_i.astype(f32)
    ones_row = jnp.full((1, CH), 1.0, f32)
    ones_col = jnp.full((CH, 1), 1.0, f32)

    def row_of(col):
        return _dote(ones_row, eye * col)

    def col_of(rowv):
        return _dote(eye * rowv, ones_col)

    keys = _f2key(sc_ref[...])
    candu = jnp.int32(0)
    for bit in range(31, -1, -1):
        m = INT_MIN if bit == 31 else jnp.int32(1 << bit)
        trial = candu | m
        scand = trial ^ INT_MIN
        cnt = jnp.sum((keys >= scand).astype(f32))
        candu = jnp.where(cnt >= K, trial, candu)
    t_key = candu ^ INT_MIN

    kf = f32(K)
    m1 = (keys > t_key).astype(f32)
    m2 = (keys == t_key).astype(f32)
    p1 = jnp.dot(m1, upper, preferred_element_type=f32)
    p2 = jnp.dot(m2, upper, preferred_element_type=f32)
    ones_ch = jnp.full((CH, 1), 1.0, f32)
    s1 = jnp.dot(m1, ones_ch, preferred_element_type=f32)
    s2 = jnp.dot(m2, ones_ch, preferred_element_type=f32)
    rc = lax.broadcasted_iota(jnp.int32, (NCH, NCH), 0)
    cc = lax.broadcasted_iota(jnp.int32, (NCH, NCH), 1)
    low_c = (cc < rc).astype(f32)
    o1 = _dote(low_c, s1)
    o2 = _dote(low_c, s2)
    n1 = jnp.sum(m1)
    pos1 = p1 + o1
    pos2 = p2 + o2 + n1
    pos_ref[...] = jnp.where(m1 > 0, pos1,
                             jnp.where((m2 > 0) & (pos2 < kf), pos2, f32(CH)))

    buf_ref[...] = jnp.zeros((CH, PAY), f32)

    def scat_body(c, carry):
        posrow = pos_ref[pl.ds(c, 1), :]
        pt = (p_col == posrow).astype(f32)
        buf_ref[...] += _dote(pt, pay_ref[c])
        return carry

    lax.fori_loop(0, NCH, scat_body, 0)

    comp = buf_ref[...]
    slot = lax.broadcasted_iota(jnp.int32, (CH, 1), 0)
    valid = slot < K
    s_col = jnp.where(valid, comp[:, 0:1], BIG_NEG)
    i_col = jnp.where(valid, comp[:, 9:10], (N + slot).astype(f32))
    s_row = row_of(s_col)
    i_row = row_of(i_col)
    gt = (s_row > s_col) | ((s_row == s_col) & (i_row < i_col))
    rank = jnp.dot(gt.astype(f32), ones_col,
                   preferred_element_type=f32)
    q = (row_of(rank) == p_col).astype(f32)
    srt = _dote(q, comp)

    aw = srt[:, 7:8]
    ah = srt[:, 8:9]
    dwc = jnp.minimum(srt[:, 3:4], CLAMP)
    dhc = jnp.minimum(srt[:, 4:5], CLAMP)
    px = srt[:, 1:2] * aw + srt[:, 5:6]
    py = srt[:, 2:3] * ah + srt[:, 6:7]
    pw = jnp.exp(dwc) * aw
    ph = jnp.exp(dhc) * ah
    bw = bounds_ref[0:1, 0:1]
    bh = bounds_ref[0:1, 1:2]
    x1 = jnp.minimum(jnp.maximum(px - 0.5 * pw, 0.0), bw)
    y1 = jnp.minimum(jnp.maximum(py - 0.5 * ph, 0.0), bh)
    x2 = jnp.minimum(jnp.maximum(px + 0.5 * pw, 0.0), bw)
    y2 = jnp.minimum(jnp.maximum(py + 0.5 * ph, 0.0), bh)
    keep0 = ((x2 - x1) >= MIN_SZ) & ((y2 - y1) >= MIN_SZ)

    x1r, y1r = row_of(x1), row_of(y1)
    x2r, y2r = row_of(x2), row_of(y2)
    area = (x2 - x1) * (y2 - y1)
    area_r = row_of(area)
    rb_n = 4
    rb_sz = CH // rb_n
    for rb in range(rb_n):
        s0 = rb * rb_sz
        xl = jnp.maximum(x1[s0:s0 + rb_sz], x1r)
        yt = jnp.maximum(y1[s0:s0 + rb_sz], y1r)
        xr = jnp.minimum(x2[s0:s0 + rb_sz], x2r)
        yb = jnp.minimum(y2[s0:s0 + rb_sz], y2r)
        inter = jnp.maximum(xr - xl, 0.0) * jnp.maximum(yb - yt, 0.0)
        union = area[s0:s0 + rb_sz] + area_r - inter
        iou = inter / (union + 1e-8)
        li = lax.broadcasted_iota(jnp.int32, (rb_sz, CH), 0) + s0
        lj = lax.broadcasted_iota(jnp.int32, (rb_sz, CH), 1)
        s_ref[s0:s0 + rb_sz, :] = ((iou > NMS_T) & (li < lj)).astype(f32)

    keep0f = keep0.astype(f32)

    def nms_cond(carry):
        return carry[1] > 0.5

    def nms_body(carry):
        kfv, _ = carry
        bad = jnp.dot(row_of(kfv), s_ref[...],
                      preferred_element_type=f32)
        nk = jnp.where(col_of(bad) > 0.5, 0.0, keep0f)
        return nk, jnp.sum(jnp.abs(nk - kfv))

    keepf, _ = lax.while_loop(nms_cond, nms_body, (keep0f, f32(1.0)))

    f_col = jnp.where(keepf > 0.5, srt[:, 0:1], f32(-1e9))
    f_row = row_of(f_col)
    pos_c = slot.astype(f32)
    pos_r = row_of(pos_c)
    gt2 = (f_row > f_col) | ((f_row == f_col) & (pos_r < pos_c))
    rank2 = jnp.dot(gt2.astype(f32), ones_col,
                    preferred_element_type=f32)
    q2 = (row_of(rank2) == p_col).astype(f32)
    m8 = jnp.concatenate(
        [x1, y1, x2, y2, f_col, jnp.zeros((CH, 3), f32)], axis=1)
    out_ref[...] = _dote(q2, m8)


def kernel(features, conv_w, conv_b, cls_w, cls_b, bbox_w, bbox_b,
           img_h, img_w):
    f32 = jnp.float32
    x = jnp.transpose(features[0], (1, 2, 0))
    x_pad = jnp.pad(x, ((1, 1), (1, 1), (0, 0)))
    w9 = jnp.transpose(conv_w, (2, 3, 1, 0)).reshape(9, 128, 128)
    head_w = jnp.concatenate(
        [cls_w[:, :, 0, 0].T, bbox_w[:, :, 0, 0].T,
         jnp.zeros((128, 1), f32)], axis=1)
    head_b = jnp.concatenate(
        [cls_b, bbox_b, jnp.zeros((1,), f32)]).reshape(1, 16)

    head = pl.pallas_call(
        _conv_head_kernel,
        out_shape=jax.ShapeDtypeStruct((HW, 16), f32),
        compiler_params=_cparams,
    )(x_pad, w9, conv_b.reshape(1, 128), head_w, head_b)

    scores = head[:, 0:3].reshape(N)
    deltas = head[:, 3:15].reshape(N, 4)

    scales = jnp.array([32.0], f32)
    ratios = jnp.array([0.5, 1.0, 2.0], f32)
    h_rat = jnp.sqrt(ratios)
    w_rat = 1.0 / h_rat
    ws = (w_rat[:, None] * scales[None, :]).reshape(-1)
    hs = (h_rat[:, None] * scales[None, :]).reshape(-1)
    cell = jnp.round(jnp.stack([-ws, -hs, ws, hs], axis=1) / 2.0)
    aw_t = cell[:, 2] - cell[:, 0]
    ah_t = cell[:, 3] - cell[:, 1]
    cx_t = cell[:, 0] + 0.5 * aw_t
    cy_t = cell[:, 1] + 0.5 * ah_t
    stride_h = jnp.asarray(img_h // H, f32)
    stride_w = jnp.asarray(img_w // W, f32)
    wg = jnp.broadcast_to(jnp.arange(W, dtype=f32)[None, :, None],
                          (H, W, A)).reshape(-1)
    hg = jnp.broadcast_to(jnp.arange(H, dtype=f32)[:, None, None],
                          (H, W, A)).reshape(-1)
    ag = jnp.broadcast_to(jnp.arange(A)[None, None, :], (H, W, A)).reshape(-1)
    cxg = wg * stride_w + cx_t[ag]
    cyg = hg * stride_h + cy_t[ag]
    awg = aw_t[ag]
    ahg = ah_t[ag]
    idxf = jnp.arange(N, dtype=f32)

    payload = jnp.concatenate(
        [scores[:, None], deltas, cxg[:, None], cyg[:, None],
         awg[:, None], ahg[:, None], idxf[:, None],
         jnp.zeros((N, 6), f32)], axis=1).reshape(NCH, CH, PAY)
    scores2d = scores.reshape(NCH, CH)
    bounds = jnp.zeros((1, 128), f32)
    bounds = bounds.at[0, 0].set(jnp.asarray(img_w, f32))
    bounds = bounds.at[0, 1].set(jnp.asarray(img_h, f32))

    final = pl.pallas_call(
        _select_nms_kernel,
        out_shape=jax.ShapeDtypeStruct((CH, 8), f32),
        scratch_shapes=[pltpu.VMEM((CH, PAY), f32),
                        pltpu.VMEM((CH, CH), f32),
                        pltpu.VMEM((NCH, CH), f32)],
        compiler_params=_cparams,
    )(scores2d, payload, bounds)

    return final[:K, 0:4], final[:K, 4]


# speedup vs baseline: 2.3323x; 1.1597x over previous
import jax
import jax.numpy as jnp
from jax import lax
from jax.experimental import pallas as pl
from jax.experimental.pallas import tpu as pltpu

H = 160
W = 160
A = 3
HW = H * W
N = HW * A
K = 1000
CH = 1024
NCH = N // CH
RT = 8
ROWS = H // RT
NMS_T = 0.7
MIN_SZ = 1e-3
CLAMP = 4.135166556742356
PAY = 16
BIG_NEG = -3.0e38
INT_MIN = -2147483648
MASK31 = 0x7FFFFFFF

_cparams = pltpu.CompilerParams(vmem_limit_bytes=120 * 1024 * 1024)


def _conv_head_kernel(x_ref, w9_ref, cb_ref, hw_ref, hb_ref, out_ref):
    cb = cb_ref[...]
    hw_m = hw_ref[...]
    hb = hb_ref[...]
    for rt in range(RT):
        acc = jnp.zeros((ROWS * W, 128), jnp.float32)
        for t in range(9):
            dh, dw = t // 3, t % 3
            sl = x_ref[rt * ROWS + dh: rt * ROWS + dh + ROWS, dw: dw + W, :]
            acc = acc + jnp.dot(sl.reshape(ROWS * W, 128), w9_ref[t],
                                preferred_element_type=jnp.float32)
        tact = jnp.maximum(acc + cb, 0.0)
        out_ref[rt * ROWS * W:(rt + 1) * ROWS * W, :] = (
            jnp.dot(tact, hw_m, preferred_element_type=jnp.float32) + hb)


def _f2key(x):
    b = lax.bitcast_convert_type(x, jnp.int32)
    return b ^ ((b >> 31) & MASK31)


def _dote(a, b):
    return jnp.dot(a, b, preferred_element_type=jnp.float32,
                   precision=lax.Precision.HIGHEST)


def _select_nms_kernel(sc_ref, pay_ref, bounds_ref, out_ref, buf_ref, s_ref,
                       pos_ref):
    f32 = jnp.float32
    r_i = lax.broadcasted_iota(jnp.int32, (CH, CH), 0)
    c_i = lax.broadcasted_iota(jnp.int32, (CH, CH), 1)
    eye = (r_i == c_i).astype(f32)
    upper = (r_i < c_i).astype(f32)
    p_col = r_i.astype(f32)
    ones_row = jnp.full((1, CH), 1.0, f32)
    ones_col = jnp.full((CH, 1), 1.0, f32)

    def row_of(col):
        return _dote(ones_row, eye * col)

    def col_of(rowv):
        return _dote(eye * rowv, ones_col)

    keys = _f2key(sc_ref[...])
    candu = jnp.int32(0)
    for bit in range(31, -1, -1):
        m = INT_MIN if bit == 31 else jnp.int32(1 << bit)
        trial = candu | m
        scand = trial ^ INT_MIN
        cnt = jnp.sum((keys >= scand).astype(f32))
        candu = jnp.where(cnt >= K, trial, candu)
    t_key = candu ^ INT_MIN

    kf = f32(K)
    m1 = (keys > t_key).astype(f32)
    m2 = (keys == t_key).astype(f32)
    p1 = jnp.dot(m1, upper, preferred_element_type=f32)
    p2 = jnp.dot(m2, upper, preferred_element_type=f32)
    ones_ch = jnp.full((CH, 1), 1.0, f32)
    s1 = jnp.dot(m1, ones_ch, preferred_element_type=f32)
    s2 = jnp.dot(m2, ones_ch, preferred_element_type=f32)
    rc = lax.broadcasted_iota(jnp.int32, (NCH, NCH), 0)
    cc = lax.broadcasted_iota(jnp.int32, (NCH, NCH), 1)
    low_c = (cc < rc).astype(f32)
    o1 = _dote(low_c, s1)
    o2 = _dote(low_c, s2)
    n1 = jnp.sum(m1)
    pos1 = p1 + o1
    pos2 = p2 + o2 + n1
    pos_ref[...] = jnp.where(m1 > 0, pos1,
                             jnp.where((m2 > 0) & (pos2 < kf), pos2, f32(CH)))

    buf_ref[...] = jnp.zeros((CH, PAY), f32)

    def scat_body(c, carry):
        posrow = pos_ref[pl.ds(c, 1), :]
        ptb = (p_col == posrow).astype(jnp.bfloat16)
        v = pay_ref[c]
        vh = v.astype(jnp.bfloat16)
        r1 = v - vh.astype(f32)
        vm = r1.astype(jnp.bfloat16)
        vl = (r1 - vm.astype(f32)).astype(jnp.bfloat16)
        buf_ref[...] += (jnp.dot(ptb, vh, preferred_element_type=f32)
                         + jnp.dot(ptb, vm, preferred_element_type=f32)
                         + jnp.dot(ptb, vl, preferred_element_type=f32))
        return carry

    lax.fori_loop(0, NCH, scat_body, 0)

    comp = buf_ref[...]
    slot = lax.broadcasted_iota(jnp.int32, (CH, 1), 0)
    valid = slot < K
    s_col = jnp.where(valid, comp[:, 0:1], BIG_NEG)
    i_col = jnp.where(valid, comp[:, 9:10], (N + slot).astype(f32))
    s_row = row_of(s_col)
    i_row = row_of(i_col)
    gt = (s_row > s_col) | ((s_row == s_col) & (i_row < i_col))
    rank = jnp.dot(gt.astype(f32), ones_col,
                   preferred_element_type=f32)
    q = (row_of(rank) == p_col).astype(f32)
    srt = _dote(q, comp)

    aw = srt[:, 7:8]
    ah = srt[:, 8:9]
    dwc = jnp.minimum(srt[:, 3:4], CLAMP)
    dhc = jnp.minimum(srt[:, 4:5], CLAMP)
    px = srt[:, 1:2] * aw + srt[:, 5:6]
    py = srt[:, 2:3] * ah + srt[:, 6:7]
    pw = jnp.exp(dwc) * aw
    ph = jnp.exp(dhc) * ah
    bw = bounds_ref[0:1, 0:1]
    bh = bounds_ref[0:1, 1:2]
    x1 = jnp.minimum(jnp.maximum(px - 0.5 * pw, 0.0), bw)
    y1 = jnp.minimum(jnp.maximum(py - 0.5 * ph, 0.0), bh)
    x2 = jnp.minimum(jnp.maximum(px + 0.5 * pw, 0.0), bw)
    y2 = jnp.minimum(jnp.maximum(py + 0.5 * ph, 0.0), bh)
    keep0 = ((x2 - x1) >= MIN_SZ) & ((y2 - y1) >= MIN_SZ)

    x1r, y1r = row_of(x1), row_of(y1)
    x2r, y2r = row_of(x2), row_of(y2)
    area = (x2 - x1) * (y2 - y1)
    area_r = row_of(area)
    rb_n = 4
    rb_sz = CH // rb_n
    for rb in range(rb_n):
        s0 = rb * rb_sz
        xl = jnp.maximum(x1[s0:s0 + rb_sz], x1r)
        yt = jnp.maximum(y1[s0:s0 + rb_sz], y1r)
        xr = jnp.minimum(x2[s0:s0 + rb_sz], x2r)
        yb = jnp.minimum(y2[s0:s0 + rb_sz], y2r)
        inter = jnp.maximum(xr - xl, 0.0) * jnp.maximum(yb - yt, 0.0)
        union = area[s0:s0 + rb_sz] + area_r - inter
        iou = inter / (union + 1e-8)
        li = lax.broadcasted_iota(jnp.int32, (rb_sz, CH), 0) + s0
        lj = lax.broadcasted_iota(jnp.int32, (rb_sz, CH), 1)
        s_ref[s0:s0 + rb_sz, :] = ((iou > NMS_T) & (li < lj)).astype(f32)

    keep0f = keep0.astype(f32)

    def nms_cond(carry):
        return carry[1] > 0.5

    def nms_body(carry):
        kfv, _ = carry
        bad = jnp.dot(row_of(kfv), s_ref[...],
                      preferred_element_type=f32)
        nk = jnp.where(col_of(bad) > 0.5, 0.0, keep0f)
        return nk, jnp.sum(jnp.abs(nk - kfv))

    keepf, _ = lax.while_loop(nms_cond, nms_body, (keep0f, f32(1.0)))

    f_col = jnp.where(keepf > 0.5, srt[:, 0:1], f32(-1e9))
    f_row = row_of(f_col)
    pos_c = slot.astype(f32)
    pos_r = row_of(pos_c)
    gt2 = (f_row > f_col) | ((f_row == f_col) & (pos_r < pos_c))
    rank2 = jnp.dot(gt2.astype(f32), ones_col,
                    preferred_element_type=f32)
    q2 = (row_of(rank2) == p_col).astype(f32)
    m8 = jnp.concatenate(
        [x1, y1, x2, y2, f_col, jnp.zeros((CH, 3), f32)], axis=1)
    out_ref[...] = _dote(q2, m8)


def kernel(features, conv_w, conv_b, cls_w, cls_b, bbox_w, bbox_b,
           img_h, img_w):
    f32 = jnp.float32
    x = jnp.transpose(features[0], (1, 2, 0))
    x_pad = jnp.pad(x, ((1, 1), (1, 1), (0, 0)))
    w9 = jnp.transpose(conv_w, (2, 3, 1, 0)).reshape(9, 128, 128)
    head_w = jnp.concatenate(
        [cls_w[:, :, 0, 0].T, bbox_w[:, :, 0, 0].T,
         jnp.zeros((128, 1), f32)], axis=1)
    head_b = jnp.concatenate(
        [cls_b, bbox_b, jnp.zeros((1,), f32)]).reshape(1, 16)

    head = pl.pallas_call(
        _conv_head_kernel,
        out_shape=jax.ShapeDtypeStruct((HW, 16), f32),
        compiler_params=_cparams,
    )(x_pad, w9, conv_b.reshape(1, 128), head_w, head_b)

    scores = head[:, 0:3].reshape(N)
    deltas = head[:, 3:15].reshape(N, 4)

    scales = jnp.array([32.0], f32)
    ratios = jnp.array([0.5, 1.0, 2.0], f32)
    h_rat = jnp.sqrt(ratios)
    w_rat = 1.0 / h_rat
    ws = (w_rat[:, None] * scales[None, :]).reshape(-1)
    hs = (h_rat[:, None] * scales[None, :]).reshape(-1)
    cell = jnp.round(jnp.stack([-ws, -hs, ws, hs], axis=1) / 2.0)
    aw_t = cell[:, 2] - cell[:, 0]
    ah_t = cell[:, 3] - cell[:, 1]
    cx_t = cell[:, 0] + 0.5 * aw_t
    cy_t = cell[:, 1] + 0.5 * ah_t
    stride_h = jnp.asarray(img_h // H, f32)
    stride_w = jnp.asarray(img_w // W, f32)
    wg = jnp.broadcast_to(jnp.arange(W, dtype=f32)[None, :, None],
                          (H, W, A)).reshape(-1)
    hg = jnp.broadcast_to(jnp.arange(H, dtype=f32)[:, None, None],
                          (H, W, A)).reshape(-1)
    ag = jnp.broadcast_to(jnp.arange(A)[None, None, :], (H, W, A)).reshape(-1)
    cxg = wg * stride_w + cx_t[ag]
    cyg = hg * stride_h + cy_t[ag]
    awg = aw_t[ag]
    ahg = ah_t[ag]
    idxf = jnp.arange(N, dtype=f32)

    payload = jnp.concatenate(
        [scores[:, None], deltas, cxg[:, None], cyg[:, None],
         awg[:, None], ahg[:, None], idxf[:, None],
         jnp.zeros((N, 6), f32)], axis=1).reshape(NCH, CH, PAY)
    scores2d = scores.reshape(NCH, CH)
    bounds = jnp.zeros((1, 128), f32)
    bounds = bounds.at[0, 0].set(jnp.asarray(img_w, f32))
    bounds = bounds.at[0, 1].set(jnp.asarray(img_h, f32))

    final = pl.pallas_call(
        _select_nms_kernel,
        out_shape=jax.ShapeDtypeStruct((CH, 8), f32),
        scratch_shapes=[pltpu.VMEM((CH, PAY), f32),
                        pltpu.VMEM((CH, CH), f32),
                        pltpu.VMEM((NCH, CH), f32)],
        compiler_params=_cparams,
    )(scores2d, payload, bounds)

    return final[:K, 0:4], final[:K, 4]
